# baseline (speedup 1.0000x reference)
import jax
import jax.numpy as jnp
import numpy as np
from jax.experimental import pallas as pl
from jax.experimental.pallas import tpu as pltpu

PI = float(np.pi)
MIN_DIST = 0.01
MIN_VAL = MIN_DIST / 10.0
HIGHEST = jax.lax.Precision.HIGHEST


def _mm(a, b):
    return jax.lax.dot_general(a, b, (((a.ndim - 1,), (0,)), ((), ())),
                               precision=HIGHEST)


def _gsum(x):
    t = x + pltpu.roll(x, 64, 1)
    return t + pltpu.roll(t, 32, 1)


_ASIN = (0.1666162484387946, 0.07795126339723928, -0.0012354098526086619,
         0.2908513674689776, -0.46127865206272023)
_SIN = (0.999999997000454, -0.16666659969977798, 0.008333097548004268,
        -0.0001981248476825909, 2.612900350327724e-06)
_PI_HI = 3.140625
_PI_LO = PI - _PI_HI


def _acos(z):
    z = jnp.clip(z, -1.0, 1.0)
    az = jnp.abs(z)
    small = az <= 0.5
    t = jnp.where(small, z * z, (1.0 - az) * 0.5)
    x = jnp.where(small, az, jnp.sqrt(t))
    r = jnp.float32(_ASIN[4])
    for cc in _ASIN[3::-1]:
        r = r * t + cc
    s = x + x * t * r
    sgn_s = jnp.where(z >= 0.0, s, -s)
    return jnp.where(small, PI / 2 - sgn_s,
                     jnp.where(z >= 0.0, 2.0 * s, PI - 2.0 * s))


def _sin_bounded(x):
    k = jnp.round(x * (1.0 / PI))
    xp = (x - k * _PI_HI) - k * _PI_LO
    t = xp * xp
    p = jnp.float32(_SIN[4])
    for cc in _SIN[3::-1]:
        p = p * t + cc
    s = xp * p
    odd = (k.astype(jnp.int32) & 1) != 0
    return jnp.where(odd, -s, s)


def _proj_kernel(sums_ref, p_ref, t16_ref, off_ref, bb_ref, cosphi_ref,
                 cf_ref, mb_ref, x_ref, c8_ref, o_ref):
    nd = 6
    P = p_ref[...]
    T16 = t16_ref[...]
    OFF = off_ref[...]
    BB = bb_ref[...]
    cosphi = cosphi_ref[...]
    CF = cf_ref[...]
    MB = mb_ref[...]
    w0, w1, w2, w3, w4, ampsum = (sums_ref[0, i] for i in range(6))
    aw = [sums_ref[0, 6 + i] for i in range(nd)]

    xs = [x_ref[:, d * 128:(d + 1) * 128] for d in range(nd)]

    x_offset = _gsum(xs[0]) * 0.25

    dpre = w0 * xs[1] + w1 * xs[2] + w2 * xs[3] + w3 * xs[4] + w4 * xs[5]
    e = jnp.exp(dpre)
    dw = e * (1.0 / _gsum(e))

    c = jnp.clip(_gsum(dw * cosphi), -1.0, 1.0)
    s = jnp.sqrt((1.0 - c) * (1.0 + c))

    r = _mm(dw, P)
    zs = [xs[d] * r for d in range(nd)]
    xw = [_gsum(z) for z in zs]
    g = jax.lax.broadcasted_iota(jnp.int32, (1, 128), 1) // 32
    xwa = jnp.where(g == 0, xw[0],
                    jnp.where(g == 1, xw[1],
                              jnp.where(g == 2, xw[2], xw[3])))
    xwb = jnp.where(g == 0, xw[4], xw[5])
    ewa = jnp.exp(xwa)
    ewb = jnp.exp(xwb)
    es = _gsum(ewa + ewb * MB)
    wl_num = _gsum(ewa * CF[0] + ewb * CF[1])
    wl = jax.nn.sigmoid(wl_num / es) + MIN_VAL
    zamp = aw[0] * zs[0]
    for d in range(1, nd):
        zamp = zamp + aw[d] * zs[d]
    amp = _gsum(zamp) - ampsum * x_offset

    ang = _mm(c8_ref[...], T16) + OFF
    sc = _sin_bounded(ang)
    six = _mm(sc, BB)
    sl1 = six[:, 0:128]
    sl2 = six[:, 128:256]
    sdlon = six[:, 256:384]
    cl1 = six[:, 384:512]
    cl2 = six[:, 512:640]
    cdlon = six[:, 640:768]
    u = cl1 * cl2
    v = sl1 * cl2
    cosd = sl1 * sl2 + u * cdlon
    dist = _acos(cosd)
    y = sdlon * cl2
    x = cl1 * sl2 - v * cdlon
    r2 = x * x + y * y
    pos = r2 > 0.0
    rinv = jnp.where(pos, jax.lax.rsqrt(r2), 0.0)
    cphi = jnp.where(pos, x * rinv, 1.0)
    sphi = y * rinv

    cdp = c * sphi - s * cphi
    arg = ((2.0 * PI) / wl) * cdp * dist
    o_ref[...] = amp * jnp.cos(arg) + x_offset


@jax.jit
def kernel(x_nh, output_coords, W_amp, W_wl, dist_weights_phi, dists_0, phi_0):
    b, n, nd, nvm, f = x_nh.shape
    bn = b * n
    k4 = output_coords.shape[-1] // bn
    lanes = nvm * f

    x = x_nh.reshape(bn, nd * lanes)
    coords8 = jax.lax.transpose(output_coords.reshape(2, bn, k4),
                                (1, 0, 2)).reshape(bn, 2 * k4)

    w_soft = jax.nn.softmax(dist_weights_phi)
    coeff = dists_0[:, 0] * W_wl[0]
    ampw = W_amp[0]
    sums = jnp.concatenate([w_soft, jnp.sum(ampw)[None], ampw]).reshape(1, 12)
    cosphi128 = jnp.repeat(jnp.cos(phi_0[0]), f).reshape(1, lanes)

    j = np.arange(lanes)
    grp = jnp.asarray((j // f).astype(np.int32))
    P = np.zeros((lanes, lanes), np.float32)
    P[(j % nvm) * f + j // nvm, j] = 1.0
    CF = jnp.stack([coeff[grp], jnp.where(grp < 2, coeff[grp + 4], 0.0)]
                   ).reshape(2, 1, lanes)
    MB = (np.asarray(j // f < 2, np.float32)).reshape(1, lanes)

    T16 = np.zeros((8, 16), np.float32)
    for k in range(4):
        T16[k, k] = 1.0
        T16[4 + k, 4 + k] = 1.0
        T16[4, 4 + k] -= 1.0
    T16[:, 8:16] = T16[:, 0:8]
    OFF = np.zeros((1, 16), np.float32)
    OFF[0, 8:16] = np.float32(PI / 2)
    BB = np.zeros((16, 6 * lanes), np.float32)
    BB[0, 0 * lanes + j] = 1.0
    BB[j // f, 1 * lanes + j] = 1.0
    BB[4 + j // f, 2 * lanes + j] = 1.0
    BB[8, 3 * lanes + j] = 1.0
    BB[8 + j // f, 4 * lanes + j] = 1.0
    BB[12 + j // f, 5 * lanes + j] = 1.0

    blk = 8
    for cand in range(min(200, bn), 0, -1):
        if bn % cand == 0 and cand % 8 == 0:
            blk = cand
            break
    grid = (bn // blk,)

    smem = pl.BlockSpec(memory_space=pltpu.SMEM)

    def const_spec(shape):
        nd_ = len(shape)
        return pl.BlockSpec(shape, lambda i, _n=nd_: (0,) * _n)

    out = pl.pallas_call(
        _proj_kernel,
        grid=grid,
        in_specs=[
            smem,
            const_spec((lanes, lanes)),
            const_spec((8, 16)),
            const_spec((1, 16)),
            const_spec((16, 6 * lanes)),
            const_spec((1, lanes)),
            const_spec((2, 1, lanes)),
            const_spec((1, lanes)),
            pl.BlockSpec((blk, nd * lanes), lambda i: (i, 0)),
            pl.BlockSpec((blk, 2 * k4), lambda i: (i, 0)),
        ],
        out_specs=pl.BlockSpec((blk, k4 * f), lambda i: (i, 0)),
        out_shape=jax.ShapeDtypeStruct((bn, k4 * f), x.dtype),
    )(sums, P, T16, np.asarray(OFF), BB, cosphi128, CF, MB, x, coords8)
    return out.reshape(b, n * k4, f)


# speedup vs baseline: 1.2193x; 1.2193x over previous
import jax
import jax.numpy as jnp
import numpy as np
from jax.experimental import pallas as pl
from jax.experimental.pallas import tpu as pltpu

PI = float(np.pi)
MIN_DIST = 0.01
MIN_VAL = MIN_DIST / 10.0
HIGHEST = jax.lax.Precision.HIGHEST


def _mm(a, b):
    return jax.lax.dot_general(a, b, (((a.ndim - 1,), (0,)), ((), ())),
                               precision=HIGHEST)


def _gsum(x):
    t = x + pltpu.roll(x, 64, 1)
    return t + pltpu.roll(t, 32, 1)


_ASIN = (0.1666162484387946, 0.07795126339723928, -0.0012354098526086619,
         0.2908513674689776, -0.46127865206272023)
_SIN = (0.999999997000454, -0.16666659969977798, 0.008333097548004268,
        -0.0001981248476825909, 2.612900350327724e-06)
_PI_HI = 3.140625
_PI_LO = PI - _PI_HI


def _acos(z):
    z = jnp.clip(z, -1.0, 1.0)
    az = jnp.abs(z)
    small = az <= 0.5
    t = jnp.where(small, z * z, (1.0 - az) * 0.5)
    x = jnp.where(small, az, jnp.sqrt(t))
    r = jnp.float32(_ASIN[4])
    for cc in _ASIN[3::-1]:
        r = r * t + cc
    s = x + x * t * r
    sgn_s = jnp.where(z >= 0.0, s, -s)
    return jnp.where(small, PI / 2 - sgn_s,
                     jnp.where(z >= 0.0, 2.0 * s, PI - 2.0 * s))


def _sin_bounded(x):
    k = jnp.round(x * (1.0 / PI))
    xp = (x - k * _PI_HI) - k * _PI_LO
    t = xp * xp
    p = jnp.float32(_SIN[4])
    for cc in _SIN[3::-1]:
        p = p * t + cc
    s = xp * p
    odd = (k.astype(jnp.int32) & 1) != 0
    return jnp.where(odd, -s, s)


def _proj_kernel(sums_ref, p_ref, t16_ref, off_ref, bb_ref, cosphi_ref,
                 cf_ref, mb_ref, x_ref, c8_ref, o_ref):
    nd = 6
    P = p_ref[...]
    T16 = t16_ref[...]
    OFF = off_ref[...]
    BB = bb_ref[...]
    cosphi = cosphi_ref[...]
    CF = cf_ref[...]
    MB = mb_ref[...]
    w0, w1, w2, w3, w4, ampsum = (sums_ref[0, i] for i in range(6))
    aw = [sums_ref[0, 6 + i] for i in range(nd)]

    xs = [x_ref[:, d * 128:(d + 1) * 128] for d in range(nd)]

    x_offset = _gsum(xs[0]) * 0.25

    dpre = w0 * xs[1] + w1 * xs[2] + w2 * xs[3] + w3 * xs[4] + w4 * xs[5]
    e = jnp.exp(dpre)
    dw = e * (1.0 / _gsum(e))

    c = jnp.clip(_gsum(dw * cosphi), -1.0, 1.0)
    s = jnp.sqrt((1.0 - c) * (1.0 + c))

    r = _mm(dw, P)
    zs = [xs[d] * r for d in range(nd)]
    xw = [_gsum(z) for z in zs]
    g = jax.lax.broadcasted_iota(jnp.int32, (1, 128), 1) // 32
    xwa = jnp.where(g == 0, xw[0],
                    jnp.where(g == 1, xw[1],
                              jnp.where(g == 2, xw[2], xw[3])))
    xwb = jnp.where(g == 0, xw[4], xw[5])
    ewa = jnp.exp(xwa)
    ewb = jnp.exp(xwb)
    es = _gsum(ewa + ewb * MB)
    wl_num = _gsum(ewa * CF[0] + ewb * CF[1])
    wl = jax.nn.sigmoid(wl_num / es) + MIN_VAL
    zamp = aw[0] * zs[0]
    for d in range(1, nd):
        zamp = zamp + aw[d] * zs[d]
    amp = _gsum(zamp) - ampsum * x_offset

    ang = _mm(c8_ref[...], T16) + OFF
    sc = _sin_bounded(ang)
    six = _mm(sc, BB)
    sl1 = six[:, 0:128]
    sl2 = six[:, 128:256]
    sdlon = six[:, 256:384]
    cl1 = six[:, 384:512]
    cl2 = six[:, 512:640]
    cdlon = six[:, 640:768]
    u = cl1 * cl2
    v = sl1 * cl2
    cosd = sl1 * sl2 + u * cdlon
    dist = _acos(cosd)
    y = sdlon * cl2
    x = cl1 * sl2 - v * cdlon
    r2 = x * x + y * y
    pos = r2 > 0.0
    rinv = jnp.where(pos, jax.lax.rsqrt(r2), 0.0)
    cphi = jnp.where(pos, x * rinv, 1.0)
    sphi = y * rinv

    cdp = c * sphi - s * cphi
    arg = ((2.0 * PI) / wl) * cdp * dist
    o_ref[...] = amp * jnp.cos(arg) + x_offset


@jax.jit
def kernel(x_nh, output_coords, W_amp, W_wl, dist_weights_phi, dists_0, phi_0):
    b, n, nd, nvm, f = x_nh.shape
    bn = b * n
    k4 = output_coords.shape[-1] // bn
    lanes = nvm * f

    x = x_nh.reshape(bn, nd * lanes)
    coords8 = jax.lax.transpose(output_coords.reshape(2, bn, k4),
                                (1, 0, 2)).reshape(bn, 2 * k4)

    w_soft = jax.nn.softmax(dist_weights_phi)
    coeff = dists_0[:, 0] * W_wl[0]
    ampw = W_amp[0]
    sums = jnp.concatenate([w_soft, jnp.sum(ampw)[None], ampw]).reshape(1, 12)
    cosphi128 = jnp.repeat(jnp.cos(phi_0[0]), f).reshape(1, lanes)

    j = np.arange(lanes)
    grp = jnp.asarray((j // f).astype(np.int32))
    P = np.zeros((lanes, lanes), np.float32)
    P[(j % nvm) * f + j // nvm, j] = 1.0
    CF = jnp.stack([coeff[grp], jnp.where(grp < 2, coeff[grp + 4], 0.0)]
                   ).reshape(2, 1, lanes)
    MB = (np.asarray(j // f < 2, np.float32)).reshape(1, lanes)

    T16 = np.zeros((8, 16), np.float32)
    for k in range(4):
        T16[k, k] = 1.0
        T16[4 + k, 4 + k] = 1.0
        T16[4, 4 + k] -= 1.0
    T16[:, 8:16] = T16[:, 0:8]
    OFF = np.zeros((1, 16), np.float32)
    OFF[0, 8:16] = np.float32(PI / 2)
    BB = np.zeros((16, 6 * lanes), np.float32)
    BB[0, 0 * lanes + j] = 1.0
    BB[j // f, 1 * lanes + j] = 1.0
    BB[4 + j // f, 2 * lanes + j] = 1.0
    BB[8, 3 * lanes + j] = 1.0
    BB[8 + j // f, 4 * lanes + j] = 1.0
    BB[12 + j // f, 5 * lanes + j] = 1.0

    blk = 8
    for cand in range(min(400, bn), 0, -1):
        if bn % cand == 0 and cand % 8 == 0:
            blk = cand
            break
    grid = (bn // blk,)

    smem = pl.BlockSpec(memory_space=pltpu.SMEM)

    def const_spec(shape):
        nd_ = len(shape)
        return pl.BlockSpec(shape, lambda i, _n=nd_: (0,) * _n)

    out = pl.pallas_call(
        _proj_kernel,
        grid=grid,
        in_specs=[
            smem,
            const_spec((lanes, lanes)),
            const_spec((8, 16)),
            const_spec((1, 16)),
            const_spec((16, 6 * lanes)),
            const_spec((1, lanes)),
            const_spec((2, 1, lanes)),
            const_spec((1, lanes)),
            pl.BlockSpec((blk, nd * lanes), lambda i: (i, 0)),
            pl.BlockSpec((blk, 2 * k4), lambda i: (i, 0)),
        ],
        out_specs=pl.BlockSpec((blk, k4 * f), lambda i: (i, 0)),
        out_shape=jax.ShapeDtypeStruct((bn, k4 * f), x.dtype),
    )(sums, P, T16, np.asarray(OFF), BB, cosphi128, CF, MB, x, coords8)
    return out.reshape(b, n * k4, f)
